# baseline (speedup 1.0000x reference)
import functools

import jax
import jax.numpy as jnp
from jax import lax
from jax.experimental import pallas as pl
from jax.experimental.pallas import tpu as pltpu
from jax.experimental.pallas import tpu_sc as plsc

_LANES = 128
_WIN = 8
_TE = 2048
_TN = 2048
_HALF = 16
_NSUB = 16
_EPAD = 1605632
_NPAD = 102400


def _sc_gather(table, idx1d):
    etot = idx1d.shape[0]
    blocks = etot // _LANES // (2 * _NSUB)
    mesh = plsc.VectorSubcoreMesh(core_axis_name="c", subcore_axis_name="s")

    @functools.partial(
        pl.kernel,
        out_type=jax.ShapeDtypeStruct((etot, _LANES), jnp.float32),
        mesh=mesh,
        scratch_types=[pltpu.VMEM((_LANES,), jnp.int32),
                       pltpu.VMEM((_LANES, _LANES), jnp.float32),
                       pltpu.SemaphoreType.DMA],
    )
    def k(tab_hbm, idx_hbm, out_hbm, idx_v, buf, sem):
        c = lax.axis_index("c")
        s = lax.axis_index("s")
        wid = c * _NSUB + s

        @pl.loop(0, blocks)
        def _(t):
            g = wid * blocks + t
            pltpu.sync_copy(idx_hbm.at[pl.ds(g * _LANES, _LANES)], idx_v)
            pltpu.async_copy(tab_hbm.at[idx_v], buf, sem).wait()
            pltpu.sync_copy(buf, out_hbm.at[pl.ds(g * _LANES, _LANES)])

    return k(table, idx1d)


def _sc_scatter_add(msg2d, idx1d, n):
    E = idx1d.shape[0]
    R = E // _LANES
    rows_per_sub = n // _NSUB
    zr = 64
    mesh = plsc.VectorSubcoreMesh(core_axis_name="c", subcore_axis_name="s")

    blocks = R // _NSUB

    @functools.partial(
        pl.kernel,
        out_type=jax.ShapeDtypeStruct((2 * n, _HALF), jnp.float32),
        mesh=mesh,
        scratch_types=[pltpu.VMEM_SHARED((n, _HALF), jnp.float32),
                       pltpu.VMEM((zr, _HALF), jnp.float32),
                       pltpu.VMEM((_LANES,), jnp.int32),
                       pltpu.VMEM((_LANES, _HALF), jnp.float32)],
    )
    def k(msg_hbm, idx_hbm, out_hbm, agg_sh, zbuf, idx_v, buf):
        c = lax.axis_index("c")
        s = lax.axis_index("s")

        @pl.loop(0, zr)
        def _(i):
            zbuf[i] = jnp.zeros((_HALF,), jnp.float32)

        @pl.loop(0, rows_per_sub // zr)
        def _(j):
            pltpu.sync_copy(
                zbuf, agg_sh.at[pl.ds(s * rows_per_sub + j * zr, zr)])

        plsc.subcore_barrier()

        @pl.loop(0, blocks)
        def _(t):
            g = s * blocks + t
            pltpu.sync_copy(idx_hbm.at[pl.ds(g * _LANES, _LANES)], idx_v)
            pltpu.sync_copy(
                msg_hbm.at[pl.ds(c * E + g * _LANES, _LANES)], buf)
            pltpu.sync_copy(buf, agg_sh.at[idx_v], add=True)

        plsc.subcore_barrier()
        pltpu.sync_copy(
            agg_sh.at[pl.ds(s * rows_per_sub, rows_per_sub)],
            out_hbm.at[pl.ds(c * n + s * rows_per_sub, rows_per_sub)])

    return k(msg2d, idx1d)


def _tc_msg(ea, xg, d_in, w1, b1, w2r_bf, b2r):
    E, EF = ea.shape
    P = d_in * 32
    te = _TE if P <= 512 else 512
    grid = E // te

    def body(ea_ref, xg_ref, w1_ref, b1_ref, w2_ref, b2_ref, out_ref):
        xg_t = xg_ref[:, :d_in]
        h = jnp.maximum(
            jnp.dot(ea_ref[...], w1_ref[...],
                    preferred_element_type=jnp.float32) + b1_ref[...], 0.0)
        t = jnp.dot(h.astype(jnp.bfloat16), w2_ref[...],
                    preferred_element_type=jnp.float32) + b2_ref[...]
        xge = jnp.repeat(xg_t, 32, axis=1)
        p = xge * t
        w = P
        while w > 32:
            w //= 2
            p = p[:, :w] + p[:, w:2 * w]
        out_ref[...] = p[:, :32]

    return pl.pallas_call(
        body,
        grid=(grid,),
        in_specs=[
            pl.BlockSpec((te, EF), lambda i: (i, 0)),
            pl.BlockSpec((te, _LANES), lambda i: (i, 0)),
            pl.BlockSpec((EF, 32), lambda i: (0, 0)),
            pl.BlockSpec((1, 32), lambda i: (0, 0)),
            pl.BlockSpec((32, P), lambda i: (0, 0)),
            pl.BlockSpec((1, P), lambda i: (0, 0)),
        ],
        out_specs=pl.BlockSpec((te, 32), lambda i: (i, 0)),
        out_shape=jax.ShapeDtypeStruct((E, 32), jnp.float32),
    )(ea, xg, w1, b1, w2r_bf, b2r)


def _tc_combine(agg2, x128, root, bias):
    d_in = root.shape[0]
    n = x128.shape[0]
    grid = n // _TN

    def body(lo_ref, hi_ref, x_ref, r_ref, b_ref, o_ref):
        x_in = x_ref[:, :d_in]
        agg = jnp.concatenate([lo_ref[0], hi_ref[0]], axis=1)
        res = jnp.maximum(
            agg + jnp.dot(x_in, r_ref[...],
                          preferred_element_type=jnp.float32) + b_ref[...],
            0.0)
        o_ref[...] = jnp.concatenate(
            [res, jnp.zeros((_TN, _LANES - 32), jnp.float32)], axis=1)

    return pl.pallas_call(
        body,
        grid=(grid,),
        in_specs=[
            pl.BlockSpec((1, _TN, _HALF), lambda i: (0, i, 0)),
            pl.BlockSpec((1, _TN, _HALF), lambda i: (1, i, 0)),
            pl.BlockSpec((_TN, _LANES), lambda i: (i, 0)),
            pl.BlockSpec((d_in, 32), lambda i: (0, 0)),
            pl.BlockSpec((1, 32), lambda i: (0, 0)),
        ],
        out_specs=pl.BlockSpec((_TN, _LANES), lambda i: (i, 0)),
        out_shape=jax.ShapeDtypeStruct((n, _LANES), jnp.float32),
    )(agg2, agg2, x128, root, bias)


def kernel(x, edge_index, edge_attr, batch,
           mlp1_w1, mlp1_b1, mlp1_w2, mlp1_b2, root1, bias1,
           mlp2_w1, mlp2_b1, mlp2_w2, mlp2_b2, root2, bias2):
    del batch
    n, nf = x.shape
    e = edge_attr.shape[0]
    dim = root1.shape[1]
    nfp = 16
    epad = _EPAD - e

    src = jnp.concatenate(
        [edge_index[0].astype(jnp.int32), jnp.zeros((epad,), jnp.int32)])
    dst = jnp.concatenate(
        [edge_index[1].astype(jnp.int32),
         n + (jnp.arange(epad, dtype=jnp.int32) % 2048)])
    eap = jnp.pad(edge_attr, ((0, epad), (0, 0)))

    xpad = jnp.pad(x, ((0, _NPAD - n), (0, _LANES - nf)))

    w2r1 = mlp1_w2.reshape(dim, nf, dim)
    w2r1 = jnp.pad(w2r1, ((0, 0), (0, nfp - nf), (0, 0))).reshape(dim, nfp * dim)
    b2r1 = jnp.pad(mlp1_b2.reshape(nf, dim),
                   ((0, nfp - nf), (0, 0))).reshape(1, nfp * dim)
    root1p = jnp.pad(root1, ((0, nfp - nf), (0, 0)))

    w2r1_bf = w2r1.astype(jnp.bfloat16)
    w2r2_bf = mlp2_w2.astype(jnp.bfloat16)
    b2r2 = mlp2_b2.reshape(1, dim * dim)

    def layer(xin128, d_in, w1, b1r, w2bf, b2r, rootp, biasr):
        xg = _sc_gather(xin128, src)
        msg = _tc_msg(eap, xg, d_in, w1, b1r, w2bf, b2r)
        aggc = jax.ops.segment_sum(msg, dst, num_segments=_NPAD)
        agg2 = jnp.stack([aggc[:, :_HALF], aggc[:, _HALF:]])
        return _tc_combine(agg2, xin128, rootp, biasr)

    x1 = layer(xpad, nfp, mlp1_w1, mlp1_b1.reshape(1, dim), w2r1_bf, b2r1,
               root1p, bias1.reshape(1, dim))
    x2 = layer(x1, dim, mlp2_w1, mlp2_b1.reshape(1, dim), w2r2_bf, b2r2,
               root2, bias2.reshape(1, dim))
    return x2[:n, :dim]
